# Initial kernel scaffold; baseline (speedup 1.0000x reference)
import jax
import jax.numpy as jnp
from jax.experimental import pallas as pl


def kernel(source_nodes, destination_nodes, negative_nodes, edge_times, edge_idxs, neighbor_node_ids, neighbor_edge_idxs, neighbor_times, memory, edge_feat_table, time_w, time_b, Wq, Wk, Wv, Wo, merge_w1, merge_b1, merge_w2, merge_b2, aff_w1, aff_b1, aff_w2, aff_b2, msg_w, msg_b, gru_wi, gru_wh, gru_bi, gru_bh):
    raise NotImplementedError("write your pallas kernel here")


# speedup vs baseline: 4.2814x; 4.2814x over previous
import functools
import math

import jax
import jax.numpy as jnp
from jax import lax
from jax.experimental import pallas as pl
from jax.experimental.pallas import tpu as pltpu
from jax.experimental.pallas import tpu_sc as plsc

N_NODES = 100000
N_EDGES = 3200000
D = 128
D_EDGE = 16
N_HEADS = 2
B = 4096
K = 10
MSG_DIM = 100

R = 256
GRID = B // R


def _dense_body(nf_ref, ngh_ref, nge_ref, nt_ref, et_ref, ef_ref,
                pn_row_ref, pn_col_ref,
                tw_ref, tb_ref,
                wq_ref, wk_a, wk_b, wk_c, wv_a, wv_b, wv_c,
                wo_a, wo_b,
                m1_a, m1_b, m1_c, m1bias, m2_ref, m2bias,
                aw1_a, aw1_b, ab1_ref, aw2_ref, ab2_ref,
                mw_a, mw_b, mw_c, mw_d, mb_ref,
                gwi_ref, gwh_ref, gbi_ref, gbh_ref,
                pos_ref, neg_ref, nm_ref, last_ref,
                vv_ref):
    i = pl.program_id(0)
    f32 = jnp.float32
    scale = f32(1.0 / math.sqrt(D))

    tw = tw_ref[...]
    tb = tb_ref[...]
    cosb = jnp.cos(tb)
    q_const = jnp.dot(cosb, wq_ref[...][D:, :], preferred_element_type=f32)

    et = et_ref[...]

    embs = []
    for s in range(3):
        nf = nf_ref[s]
        q = (jnp.dot(nf, wq_ref[...][:D, :], preferred_element_type=f32)
             + q_const) * scale
        l0 = []
        l1 = []
        for k in range(K):
            ngh = ngh_ref[k, s]
            nge = nge_ref[k, s]
            dt = et - nt_ref[k, s]
            tf = jnp.cos(dt * tw + tb)
            kk = (jnp.dot(ngh, wk_a[...], preferred_element_type=f32)
                  + jnp.dot(nge, wk_b[...], preferred_element_type=f32)
                  + jnp.dot(tf, wk_c[...], preferred_element_type=f32))
            vv = (jnp.dot(ngh, wv_a[...], preferred_element_type=f32)
                  + jnp.dot(nge, wv_b[...], preferred_element_type=f32)
                  + jnp.dot(tf, wv_c[...], preferred_element_type=f32))
            vv_ref[k] = vv
            l0.append(jnp.sum(q[:, :D] * kk[:, :D], axis=1, keepdims=True))
            l1.append(jnp.sum(q[:, D:] * kk[:, D:], axis=1, keepdims=True))
        m0 = functools.reduce(jnp.maximum, l0)
        m1 = functools.reduce(jnp.maximum, l1)
        ao0 = jnp.zeros((R, D), f32)
        ao1 = jnp.zeros((R, D), f32)
        s0 = jnp.zeros((R, 1), f32)
        s1 = jnp.zeros((R, 1), f32)
        for k in range(K):
            vv = vv_ref[k]
            w0 = jnp.exp(l0[k] - m0)
            w1 = jnp.exp(l1[k] - m1)
            ao0 = ao0 + w0 * vv[:, :D]
            ao1 = ao1 + w1 * vv[:, D:]
            s0 = s0 + w0
            s1 = s1 + w1
        ao0 = ao0 / s0
        ao1 = ao1 / s1
        ao = (jnp.dot(ao0, wo_a[...], preferred_element_type=f32)
              + jnp.dot(ao1, wo_b[...], preferred_element_type=f32))
        h1 = (jnp.dot(ao[:, :D], m1_a[...], preferred_element_type=f32)
              + jnp.dot(ao[:, D:], m1_b[...], preferred_element_type=f32)
              + jnp.dot(nf, m1_c[...], preferred_element_type=f32)
              + m1bias[...])
        h1 = jnp.maximum(h1, 0.0)
        emb = jnp.dot(h1, m2_ref[...], preferred_element_type=f32) + m2bias[...]
        embs.append(emb)

    def aff(a, b_):
        x = (jnp.dot(a, aw1_a[...], preferred_element_type=f32)
             + jnp.dot(b_, aw1_b[...], preferred_element_type=f32)
             + ab1_ref[...])
        x = jnp.maximum(x, 0.0)
        y = jnp.dot(x, aw2_ref[...], preferred_element_type=f32) + ab2_ref[...]
        return 1.0 / (1.0 + jnp.exp(-y))
    pos_ref[...] = aff(embs[0], embs[1])
    neg_ref[...] = aff(embs[0], embs[2])

    ef = ef_ref[...]
    tfe = jnp.cos(et * tw + tb)
    nf0 = nf_ref[0]
    nf1 = nf_ref[1]
    for half, (a, b_) in enumerate(((nf0, nf1), (nf1, nf0))):
        msg = (jnp.dot(a, mw_a[...], preferred_element_type=f32)
               + jnp.dot(b_, mw_b[...], preferred_element_type=f32)
               + jnp.dot(ef, mw_c[...], preferred_element_type=f32)
               + jnp.dot(tfe, mw_d[...], preferred_element_type=f32)
               + mb_ref[...])
        msg = jnp.maximum(msg, 0.0)
        gi = jnp.dot(msg, gwi_ref[...], preferred_element_type=f32) + gbi_ref[...]
        gh = jnp.dot(a, gwh_ref[...], preferred_element_type=f32) + gbh_ref[...]
        r = 1.0 / (1.0 + jnp.exp(-(gi[:, :D] + gh[:, :D])))
        z = 1.0 / (1.0 + jnp.exp(-(gi[:, D:2 * D] + gh[:, D:2 * D])))
        g = jnp.tanh(gi[:, 2 * D:] + r * gh[:, 2 * D:])
        nm_ref[half] = (1.0 - z) * g + z * a

    pn_row = pn_row_ref[...]
    for half in range(2):
        mine = pn_col_ref[half]
        acc = jnp.full((R, 1), -1, jnp.int32)
        CH = 1024
        for j0 in range(0, 2 * B, CH):
            chunk = pn_row[:, j0:j0 + CH]
            jidx = lax.broadcasted_iota(jnp.int32, (R, CH), 1) + j0
            eq = mine == chunk
            acc = jnp.maximum(acc, jnp.max(jnp.where(eq, jidx, -1), axis=1,
                                           keepdims=True))
        last_ref[half] = acc


def _dense_call(nf3, ngh4, nge4, nt4, et_col, ef, pn_row, pn_col, params):
    (tw, tb, wq, wk_a, wk_b, wk_c, wv_a, wv_b, wv_c, wo_a, wo_b,
     m1_a, m1_b, m1_c, m1bias, m2, m2bias,
     aw1_a, aw1_b, ab1, aw2, ab2,
     mw_a, mw_b, mw_c, mw_d, mb, gwi, gwh, gbi, gbh) = params
    f32 = jnp.float32
    full = lambda arr: pl.BlockSpec(arr.shape, lambda i: (0,) * arr.ndim)
    in_specs = [
        pl.BlockSpec((3, R, D), lambda i: (0, i, 0)),
        pl.BlockSpec((K, 3, R, D), lambda i: (0, 0, i, 0)),
        pl.BlockSpec((K, 3, R, D_EDGE), lambda i: (0, 0, i, 0)),
        pl.BlockSpec((K, 3, R, 1), lambda i: (0, 0, i, 0)),
        pl.BlockSpec((R, 1), lambda i: (i, 0)),
        pl.BlockSpec((R, D_EDGE), lambda i: (i, 0)),
        full(pn_row),
        pl.BlockSpec((2, R, 1), lambda i: (0, i, 0)),
    ] + [full(p) for p in params]
    out_specs = [
        pl.BlockSpec((R, 1), lambda i: (i, 0)),
        pl.BlockSpec((R, 1), lambda i: (i, 0)),
        pl.BlockSpec((2, R, D), lambda i: (0, i, 0)),
        pl.BlockSpec((2, R, 1), lambda i: (0, i, 0)),
    ]
    out_shape = [
        jax.ShapeDtypeStruct((B, 1), f32),
        jax.ShapeDtypeStruct((B, 1), f32),
        jax.ShapeDtypeStruct((2, B, D), f32),
        jax.ShapeDtypeStruct((2, B, 1), jnp.int32),
    ]
    return pl.pallas_call(
        _dense_body,
        grid=(GRID,),
        in_specs=in_specs,
        out_specs=out_specs,
        out_shape=out_shape,
        scratch_shapes=[pltpu.VMEM((K, R, 2 * D), f32)],
        compiler_params=pltpu.CompilerParams(
            dimension_semantics=("arbitrary",)),
    )(nf3, ngh4, nge4, nt4, et_col, ef, pn_row, pn_col, *params)


def _prep_params(time_w, time_b, Wq, Wk, Wv, Wo, merge_w1, merge_b1,
                 merge_w2, merge_b2, aff_w1, aff_b1, aff_w2, aff_b2,
                 msg_w, msg_b, gru_wi, gru_wh, gru_bi, gru_bh):
    row = lambda v: v.reshape(1, -1)
    return (
        row(time_w), row(time_b), Wq,
        Wk[:D], Wk[D:D + D_EDGE], Wk[D + D_EDGE:],
        Wv[:D], Wv[D:D + D_EDGE], Wv[D + D_EDGE:],
        Wo[:D], Wo[D:],
        merge_w1[:D], merge_w1[D:2 * D], merge_w1[2 * D:], row(merge_b1),
        merge_w2, row(merge_b2),
        aff_w1[:D], aff_w1[D:], row(aff_b1), aff_w2, row(aff_b2),
        msg_w[:D], msg_w[D:2 * D], msg_w[2 * D:2 * D + D_EDGE],
        msg_w[2 * D + D_EDGE:], row(msg_b),
        gru_wi, gru_wh, row(gru_bi), row(gru_bh),
    )


def kernel(source_nodes, destination_nodes, negative_nodes, edge_times,
           edge_idxs, neighbor_node_ids, neighbor_edge_idxs, neighbor_times,
           memory, edge_feat_table, time_w, time_b, Wq, Wk, Wv, Wo,
           merge_w1, merge_b1, merge_w2, merge_b2,
           aff_w1, aff_b1, aff_w2, aff_b2,
           msg_w, msg_b, gru_wi, gru_wh, gru_bi, gru_bh):
    nodes = jnp.concatenate([source_nodes, destination_nodes, negative_nodes])
    pos_nodes = nodes[:2 * B]

    node_feat = jnp.take(memory, nodes, axis=0)
    ngh_ids_km = neighbor_node_ids.T.reshape(-1)
    ngh_eid_km = neighbor_edge_idxs.T.reshape(-1)
    ngh_feat = jnp.take(memory, ngh_ids_km, axis=0)
    ngh_edge = jnp.take(edge_feat_table, ngh_eid_km, axis=0)
    e_feat = jnp.take(edge_feat_table, edge_idxs, axis=0)

    nf3 = node_feat.reshape(3, B, D)
    ngh4 = ngh_feat.reshape(K, 3, B, D)
    nge4 = ngh_edge.reshape(K, 3, B, D_EDGE)
    nt4 = neighbor_times.T.reshape(K, 3, B, 1)
    et_col = edge_times.reshape(B, 1)
    pn_row = pos_nodes.reshape(1, 2 * B)
    pn_col = pos_nodes.reshape(2, B, 1)
    params = _prep_params(time_w, time_b, Wq, Wk, Wv, Wo, merge_w1, merge_b1,
                          merge_w2, merge_b2, aff_w1, aff_b1, aff_w2, aff_b2,
                          msg_w, msg_b, gru_wi, gru_wh, gru_bi, gru_bh)
    pos2, neg2, nm3, last3 = _dense_call(nf3, ngh4, nge4, nt4, et_col, e_feat,
                                         pn_row, pn_col, params)
    new_mem = nm3.reshape(2 * B, D)
    last = last3.reshape(2 * B)

    upd = jnp.take(new_mem, last, axis=0)
    updated_memory = memory.at[pos_nodes].set(upd)
    return pos2.reshape(B), neg2.reshape(B), updated_memory
